# baseline (speedup 1.0000x reference)
import functools

import jax
import jax.numpy as jnp
from jax import lax
from jax.experimental import pallas as pl
from jax.experimental.pallas import tpu as pltpu
from jax.experimental.pallas import tpu_sc as plsc

_SCALE = 8.0
_LANES = 16


@functools.cache
def _make_gather(V, D, H, B):
    info = plsc.get_sparse_core_info()
    NC, NS = info.num_cores, info.num_subcores
    NW = NC * NS
    BT = B // 128
    assert BT == NW and D == 64
    DT = D // 8
    NG = D // _LANES

    mesh = plsc.VectorSubcoreMesh(core_axis_name="c", subcore_axis_name="s")

    @functools.partial(
        pl.kernel,
        mesh=mesh,
        out_type=jax.ShapeDtypeStruct((H, DT, BT, 8, 128), jnp.float32),
        name="emb_gather_sc",
        scratch_types=[
            pltpu.VMEM((H, 128), jnp.int32),
            pltpu.VMEM((128, D), jnp.float32),
            pltpu.VMEM((128, D), jnp.float32),
            pltpu.VMEM((DT, 1, 8, 128), jnp.float32),
            pltpu.VMEM((DT, 1, 8, 128), jnp.float32),
            pltpu.SemaphoreType.DMA,
            pltpu.SemaphoreType.DMA,
            pltpu.SemaphoreType.DMA,
            pltpu.SemaphoreType.DMA,
        ],
        compiler_params=pltpu.CompilerParams(
            use_tc_tiling_on_sc=False, needs_layout_passes=False),
    )
    def gather_kernel(table_hbm, idx_hbm, out_hbm,
                      idx_v, a0, a1, b0, b1, gs0, gs1, os0, os1):
        A = (a0, a1)
        Bb = (b0, b1)
        gsem = (gs0, gs1)
        osem = (os0, os1)
        wid = lax.axis_index("s") * NC + lax.axis_index("c")

        pltpu.sync_copy(idx_hbm.at[:, pl.ds(wid * 128, 128)], idx_v)

        iot = lax.iota(jnp.int32, _LANES)
        zer = iot * 0
        hi_g = [lax.shift_right_logical(iot, 3) + 2 * g for g in range(NG)]
        ds_v = iot & 7

        def start_gather(h, buf):
            pltpu.async_copy(
                table_hbm.at[idx_v.at[h]], A[buf], gsem[buf])

        def wait_gather(h, buf):
            pltpu.make_async_copy(
                table_hbm.at[idx_v.at[h]], A[buf], gsem[buf]).wait()

        def start_out(h, buf):
            pltpu.async_copy(
                Bb[buf], out_hbm.at[h, :, pl.ds(wid, 1)], osem[buf])

        def wait_out(h, buf):
            pltpu.make_async_copy(
                Bb[buf], out_hbm.at[h, :, pl.ds(wid, 1)], osem[buf]).wait()

        start_gather(0, 0)
        start_gather(1, 1)

        def pair_body(p, carry):
            for buf in range(2):
                h = p * 2 + buf
                wait_gather(h, buf)

                @pl.when(h >= 2)
                def _():
                    wait_out(h - 2, buf)

                a = A[buf]
                bv = Bb[buf]

                @plsc.parallel_loop(0, 128, unroll=4)
                def _(bl):
                    bl_v = zer + bl
                    for g in range(NG):
                        v = a[bl, pl.ds(g * _LANES, _LANES)] * _SCALE
                        plsc.store_scatter(bv, [hi_g[g], zer, ds_v, bl_v], v)

                start_out(h, buf)

                @pl.when(h + 2 < H)
                def _():
                    start_gather(h + 2, buf)

            return carry

        lax.fori_loop(0, H // 2, pair_body, 0)
        wait_out(H - 2, 0)
        wait_out(H - 1, 1)

    return gather_kernel


def kernel(x, table):
    B, H = x.shape
    V, D = table.shape
    xT = jnp.transpose(x).astype(jnp.int32)
    img = _make_gather(V, D, H, B)(table, xT)
    out = img.transpose(2, 4, 0, 1, 3).reshape(B, H, D)
    return out


# speedup vs baseline: 1.0584x; 1.0584x over previous
import functools

import jax
import jax.numpy as jnp
from jax import lax
from jax.experimental import pallas as pl
from jax.experimental.pallas import tpu as pltpu
from jax.experimental.pallas import tpu_sc as plsc

_SCALE = 8.0
_LANES = 16


@functools.cache
def _make_gather(V, D, N):
    info = plsc.get_sparse_core_info()
    NC, NS = info.num_cores, info.num_subcores
    NW = NC * NS
    assert N % NW == 0
    RW = N // NW
    CH = 640
    assert RW % (2 * CH) == 0
    NCHUNK = RW // CH

    mesh = plsc.VectorSubcoreMesh(core_axis_name="c", subcore_axis_name="s")

    @functools.partial(
        pl.kernel,
        mesh=mesh,
        out_type=jax.ShapeDtypeStruct((N, D), jnp.float32),
        name="emb_gather_sc",
        scratch_types=[
            pltpu.VMEM((RW,), jnp.int32),
            pltpu.VMEM((CH, D), jnp.float32),
            pltpu.VMEM((CH, D), jnp.float32),
            pltpu.SemaphoreType.DMA,
            pltpu.SemaphoreType.DMA,
            pltpu.SemaphoreType.DMA,
            pltpu.SemaphoreType.DMA,
        ],
        compiler_params=pltpu.CompilerParams(use_tc_tiling_on_sc=False),
    )
    def gather_kernel(table_hbm, idx_hbm, out_hbm,
                      idx_v, rows0, rows1, gs0, gs1, os0, os1):
        rows = (rows0, rows1)
        gsem = (gs0, gs1)
        osem = (os0, os1)
        wid = lax.axis_index("s") * NC + lax.axis_index("c")
        base = wid * RW

        pltpu.sync_copy(idx_hbm.at[pl.ds(base, RW)], idx_v)

        def start_gather(c, b):
            pltpu.async_copy(
                table_hbm.at[idx_v.at[pl.ds(c * CH, CH)]], rows[b], gsem[b])

        def wait_gather(c, b):
            pltpu.make_async_copy(
                table_hbm.at[idx_v.at[pl.ds(c * CH, CH)]], rows[b],
                gsem[b]).wait()

        def start_out(c, b):
            pltpu.async_copy(
                rows[b], out_hbm.at[pl.ds(base + c * CH, CH)], osem[b])

        def wait_out(c, b):
            pltpu.make_async_copy(
                rows[b], out_hbm.at[pl.ds(base + c * CH, CH)], osem[b]).wait()

        start_gather(0, 0)
        start_gather(1, 1)

        def pair_body(p, carry):
            for b in range(2):
                c = p * 2 + b
                wait_gather(c, b)
                buf = rows[b]

                @plsc.parallel_loop(0, CH, unroll=4)
                def _(r):
                    for d in range(D // _LANES):
                        sl = (r, pl.ds(d * _LANES, _LANES))
                        buf[sl] = buf[sl] * _SCALE

                start_out(c, b)

                @pl.when(c + 2 < NCHUNK)
                def _():
                    wait_out(c, b)
                    start_gather(c + 2, b)

            return carry

        lax.fori_loop(0, NCHUNK // 2, pair_body, 0)
        wait_out(NCHUNK - 2, 0)
        wait_out(NCHUNK - 1, 1)

    return gather_kernel


def kernel(x, table):
    B, H = x.shape
    V, D = table.shape
    N = B * H
    flat = x.reshape(N).astype(jnp.int32)
    out = _make_gather(V, D, N)(table, flat)
    return out.reshape(B, H, D)


# speedup vs baseline: 1.6343x; 1.5442x over previous
import functools

import jax
import jax.numpy as jnp
from jax import lax
from jax.experimental import pallas as pl
from jax.experimental.pallas import tpu as pltpu
from jax.experimental.pallas import tpu_sc as plsc

_SCALE = 8.0
_LANES = 16
_PITCH = 129


@functools.cache
def _make_gather(V, D, H, B):
    info = plsc.get_sparse_core_info()
    NC, NS = info.num_cores, info.num_subcores
    NW = NC * NS
    BT = B // 128
    assert BT == NW and D == 64
    DT = D // 8
    NG = D // _LANES

    mesh = plsc.VectorSubcoreMesh(core_axis_name="c", subcore_axis_name="s")

    @functools.partial(
        pl.kernel,
        mesh=mesh,
        out_type=jax.ShapeDtypeStruct((H, DT, BT, 8, 128), jnp.float32),
        name="emb_gather_sc",
        scratch_types=[
            pltpu.VMEM((H, 128), jnp.int32),
            pltpu.VMEM((128, D), jnp.float32),
            pltpu.VMEM((128, D), jnp.float32),
            pltpu.VMEM((D * _PITCH,), jnp.float32),
            pltpu.VMEM((DT, 1, 8, 128), jnp.float32),
            pltpu.VMEM((DT, 1, 8, 128), jnp.float32),
            pltpu.SemaphoreType.DMA,
            pltpu.SemaphoreType.DMA,
            pltpu.SemaphoreType.DMA,
            pltpu.SemaphoreType.DMA,
        ],
        compiler_params=pltpu.CompilerParams(
            use_tc_tiling_on_sc=False, needs_layout_passes=False),
    )
    def gather_kernel(table_hbm, idx_hbm, out_hbm,
                      idx_v, a0, a1, tmp, b0, b1, gs0, gs1, os0, os1):
        A = (a0, a1)
        Bb = (b0, b1)
        gsem = (gs0, gs1)
        osem = (os0, os1)
        wid = lax.axis_index("s") * NC + lax.axis_index("c")

        pltpu.sync_copy(idx_hbm.at[:, pl.ds(wid * 128, 128)], idx_v)

        iot = lax.iota(jnp.int32, _LANES)
        base_g = [(iot + _LANES * g) * _PITCH for g in range(NG)]

        def start_gather(h, buf):
            pltpu.async_copy(
                table_hbm.at[idx_v.at[h]], A[buf], gsem[buf])

        def wait_gather(h, buf):
            pltpu.make_async_copy(
                table_hbm.at[idx_v.at[h]], A[buf], gsem[buf]).wait()

        def start_out(h, buf):
            pltpu.async_copy(
                Bb[buf], out_hbm.at[h, :, pl.ds(wid, 1)], osem[buf])

        def wait_out(h, buf):
            pltpu.make_async_copy(
                Bb[buf], out_hbm.at[h, :, pl.ds(wid, 1)], osem[buf]).wait()

        start_gather(0, 0)
        start_gather(1, 1)

        def pair_body(p, carry):
            for buf in range(2):
                h = p * 2 + buf
                wait_gather(h, buf)

                @pl.when(h >= 2)
                def _():
                    wait_out(h - 2, buf)

                a = A[buf]
                bv = Bb[buf]

                @plsc.parallel_loop(0, 128, unroll=4)
                def _(bl):
                    for g in range(NG):
                        v = a[bl, pl.ds(g * _LANES, _LANES)]
                        plsc.store_scatter(tmp, [base_g[g] + bl], v)

                @plsc.parallel_loop(0, D, unroll=4)
                def _(d):
                    dt = lax.shift_right_logical(d, 3)
                    ds = d & 7
                    row = d * _PITCH
                    for k in range(8):
                        v = tmp[pl.ds(row + k * _LANES, _LANES)] * _SCALE
                        bv[dt, 0, ds, pl.ds(k * _LANES, _LANES)] = v

                start_out(h, buf)

                @pl.when(h + 2 < H)
                def _():
                    start_gather(h + 2, buf)

            return carry

        lax.fori_loop(0, H // 2, pair_body, 0)
        wait_out(H - 2, 0)
        wait_out(H - 1, 1)

    return gather_kernel


def kernel(x, table):
    B, H = x.shape
    V, D = table.shape
    xT = jnp.transpose(x).astype(jnp.int32)
    img = _make_gather(V, D, H, B)(table, xT)
    out = img.transpose(2, 4, 0, 1, 3).reshape(B, H, D)
    return out
